# baseline (speedup 1.0000x reference)
import functools

import jax
import jax.numpy as jnp
from jax import lax
from jax.experimental import pallas as pl
from jax.experimental.pallas import tpu as pltpu
from jax.experimental.pallas import tpu_sc as plsc

V = 1000000
DIM = 64
B = 4096
S = 200
S_SUB = 100
NC, NS = 2, 16
NW = NC * NS
B_PER_W = B // NW
NBUF = 4
N_ROUND = B_PER_W // NBUF

VB = 8192
NSTEP = -(-V // VB)

_mesh = plsc.VectorSubcoreMesh(core_axis_name="c", subcore_axis_name="s")


def _linearize_body(wt_ref, out_ref):
    out_ref[:, 0:DIM] = jnp.transpose(wt_ref[...], (1, 0))


_linearize = pl.pallas_call(
    _linearize_body,
    grid=(NSTEP,),
    in_specs=[pl.BlockSpec((DIM, VB), lambda i: (0, i))],
    out_specs=pl.BlockSpec((VB, 128), lambda i: (i, 0)),
    out_shape=jax.ShapeDtypeStruct((V, 128), jnp.float32),
)


@functools.partial(
    pl.kernel,
    mesh=_mesh,
    out_type=jax.ShapeDtypeStruct((B, S_SUB, DIM), jnp.float32),
    scratch_types=[
        pltpu.VMEM((B_PER_W, S_SUB), jnp.int32),
        pltpu.VMEM((NBUF, S_SUB, 128), jnp.float32),
        [pltpu.SemaphoreType.DMA] * NBUF,
        [pltpu.SemaphoreType.DMA] * NBUF,
    ],
    compiler_params=pltpu.CompilerParams(use_tc_tiling_on_sc=False),
)
def _emb_lookup(idx_hbm, table_hbm, out_hbm, idx_v, rows_v, gsems, wsems):
    wid = lax.axis_index("s") * NC + lax.axis_index("c")
    pltpu.sync_copy(idx_hbm.at[pl.ds(wid * B_PER_W, B_PER_W)], idx_v)
    b_base = wid * B_PER_W

    def start_gather(j, b):
        pltpu.async_copy(table_hbm.at[idx_v.at[j]], rows_v.at[b], gsems[b])

    for b in range(NBUF):
        start_gather(b, b)

    def body(r, carry):
        j0 = r * NBUF
        for b in range(NBUF):
            pltpu.make_async_copy(
                table_hbm.at[idx_v.at[0]], rows_v.at[b], gsems[b]).wait()
            pltpu.async_copy(
                rows_v.at[b, :, pl.ds(0, DIM)],
                out_hbm.at[b_base + j0 + b], wsems[b])
        for b in range(NBUF):
            pltpu.make_async_copy(
                rows_v.at[b, :, pl.ds(0, DIM)], out_hbm.at[0],
                wsems[b]).wait()
            jn = j0 + b + NBUF

            @pl.when(jn < B_PER_W)
            def _():
                start_gather(jn, b)

        return carry

    lax.fori_loop(0, N_ROUND, body, 0)


def kernel(input_ids, weight):
    wt = weight.T
    table = _linearize(wt)
    halves = [_emb_lookup(input_ids[:, 0:S_SUB], table),
              _emb_lookup(input_ids[:, S_SUB:S], table)]
    out_t = jnp.concatenate(
        [jnp.transpose(h, (1, 2, 0)) for h in halves], axis=0)
    return out_t.transpose(2, 0, 1)
